# baseline (device time: 26095 ns/iter reference)
import jax
import jax.numpy as jnp
from jax import lax
from jax.experimental import pallas as pl
from jax.experimental.pallas import tpu as pltpu

C = 8
HALF = 512
CH = HALF // C


def kernel(partial, resid, gamma):
    m, d = resid.shape
    gamma2 = gamma.reshape(1, d)

    def body(partial_ref, resid_ref, gamma_ref, out_ref,
             send_buf, nb_buf, x_send, x_recv, y_send, y_recv):
        my_x = lax.axis_index("x")
        my_y = lax.axis_index("y")
        my_z = lax.axis_index("z")
        h = my_y % 2
        base = h * HALF
        other = (1 - h) * HALF
        nbr = (1 - my_x, my_y, my_z)
        partner = (my_x, my_y + 1 - 2 * h, my_z)

        barrier_sem = pltpu.get_barrier_semaphore()
        for peer in (nbr, partner):
            pl.semaphore_signal(barrier_sem, inc=1, device_id=peer,
                                device_id_type=pl.DeviceIdType.MESH)
        pl.semaphore_wait(barrier_sem, 2)

        def compute_rows(r0):
            del r0

        x_rdmas = []
        for c in range(C):
            send_buf[pl.ds(c * CH, CH), :] = (
                partial_ref[0, pl.ds(base + c * CH, CH), :]
                .astype(jnp.bfloat16))
            rdma = pltpu.make_async_remote_copy(
                src_ref=send_buf.at[pl.ds(c * CH, CH), :],
                dst_ref=nb_buf.at[pl.ds(base + c * CH, CH), :],
                send_sem=x_send.at[c],
                recv_sem=x_recv.at[c],
                device_id=nbr,
                device_id_type=pl.DeviceIdType.MESH,
            )
            rdma.start()
            x_rdmas.append(rdma)

        y_rdmas = []
        for c in range(C):
            x_rdmas[c].wait_recv()
            rdma = pltpu.make_async_remote_copy(
                src_ref=nb_buf.at[pl.ds(base + c * CH, CH), :],
                dst_ref=nb_buf.at[pl.ds(base + c * CH, CH), :],
                send_sem=y_send.at[c],
                recv_sem=y_recv.at[c],
                device_id=partner,
                device_id_type=pl.DeviceIdType.MESH,
            )
            rdma.start()
            y_rdmas.append(rdma)
            compute_rows(base + c * CH)

        for c in range(C):
            y_rdmas[c].wait_recv()
            compute_rows(other + c * CH)

        for c in range(C):
            x_rdmas[c].wait_send()
            y_rdmas[c].wait_send()
        out_ref[...] = resid_ref[...]

    return pl.pallas_call(
        body,
        out_shape=jax.ShapeDtypeStruct((m, d), jnp.float32),
        in_specs=[
            pl.BlockSpec(memory_space=pltpu.VMEM),
            pl.BlockSpec(memory_space=pltpu.VMEM),
            pl.BlockSpec(memory_space=pltpu.VMEM),
        ],
        out_specs=pl.BlockSpec(memory_space=pltpu.VMEM),
        scratch_shapes=[
            pltpu.VMEM((HALF, d), jnp.bfloat16),
            pltpu.VMEM((m, d), jnp.bfloat16),
            pltpu.SemaphoreType.DMA((C,)),
            pltpu.SemaphoreType.DMA((C,)),
            pltpu.SemaphoreType.DMA((C,)),
            pltpu.SemaphoreType.DMA((C,)),
        ],
        compiler_params=pltpu.CompilerParams(collective_id=0),
    )(partial, resid, gamma2)


# device time: 22283 ns/iter; 1.1711x vs baseline; 1.1711x over previous
import jax
import jax.numpy as jnp
from jax import lax
from jax.experimental import pallas as pl
from jax.experimental.pallas import tpu as pltpu

C = 8
HALF = 512
CH = HALF // C


def kernel(partial, resid, gamma):
    m, d = resid.shape
    gamma2 = gamma.reshape(1, d)

    def body(partial_ref, resid_ref, gamma_ref, out_ref,
             send_buf, nb_buf, x_send, x_recv):
        my_x = lax.axis_index("x")
        my_y = lax.axis_index("y")
        my_z = lax.axis_index("z")
        h = my_y % 2
        base = h * HALF
        nbr = (1 - my_x, my_y, my_z)

        barrier_sem = pltpu.get_barrier_semaphore()
        pl.semaphore_signal(barrier_sem, inc=1, device_id=nbr,
                            device_id_type=pl.DeviceIdType.MESH)
        pl.semaphore_wait(barrier_sem, 1)

        x_rdmas = []
        for c in range(C):
            send_buf[pl.ds(c * CH, CH), :] = (
                partial_ref[0, pl.ds(base + c * CH, CH), :]
                .astype(jnp.bfloat16))
            rdma = pltpu.make_async_remote_copy(
                src_ref=send_buf.at[pl.ds(c * CH, CH), :],
                dst_ref=nb_buf.at[pl.ds(base + c * CH, CH), :],
                send_sem=x_send.at[c],
                recv_sem=x_recv.at[c],
                device_id=nbr,
                device_id_type=pl.DeviceIdType.MESH,
            )
            rdma.start()
            x_rdmas.append(rdma)

        for c in range(C):
            x_rdmas[c].wait_recv()
        for c in range(C):
            x_rdmas[c].wait_send()
        out_ref[...] = resid_ref[...]

    return pl.pallas_call(
        body,
        out_shape=jax.ShapeDtypeStruct((m, d), jnp.float32),
        in_specs=[
            pl.BlockSpec(memory_space=pltpu.VMEM),
            pl.BlockSpec(memory_space=pltpu.VMEM),
            pl.BlockSpec(memory_space=pltpu.VMEM),
        ],
        out_specs=pl.BlockSpec(memory_space=pltpu.VMEM),
        scratch_shapes=[
            pltpu.VMEM((HALF, d), jnp.bfloat16),
            pltpu.VMEM((m, d), jnp.bfloat16),
            pltpu.SemaphoreType.DMA((C,)),
            pltpu.SemaphoreType.DMA((C,)),
        ],
        compiler_params=pltpu.CompilerParams(collective_id=0),
    )(partial, resid, gamma2)


# device time: 22246 ns/iter; 1.1730x vs baseline; 1.0017x over previous
import jax
import jax.numpy as jnp
from jax import lax
from jax.experimental import pallas as pl
from jax.experimental.pallas import tpu as pltpu

Q = 256
CH = 32
NF = Q // CH
DA = 96
DB = 96
DC = 64
NA = DA // CH
NB = DB // CH
NC = DC // CH
NX = NF + NA
SB = Q + DA


def kernel(partial, resid, gamma):
    m, d = resid.shape
    gamma2 = gamma.reshape(1, d)

    def body(partial_ref, resid_ref, gamma_ref, out_ref, sbuf, nb_buf,
             x_s, x_r, yf_s, yf_r, zf_s, zf_r, yr_s, yr_r, zr_s, zr_r):
        my_x = lax.axis_index("x")
        my_y = lax.axis_index("y")
        my_z = lax.axis_index("z")
        hy = my_y % 2
        hz = my_z % 2
        q_me = (2 * hy + hz) * Q
        q_y = (2 * (1 - hy) + hz) * Q
        q_z = (2 * hy + (1 - hz)) * Q
        q_d = (2 * (1 - hy) + (1 - hz)) * Q
        X = (1 - my_x, my_y, my_z)
        Y = (my_x, my_y + 1 - 2 * hy, my_z)
        Z = (my_x, my_y, my_z + 1 - 2 * hz)

        sbuf[0:Q, :] = partial_ref[0, pl.ds(q_me, Q), :].astype(jnp.bfloat16)
        sbuf[Q:SB, :] = partial_ref[0, pl.ds(q_d, DA), :].astype(jnp.bfloat16)

        barrier_sem = pltpu.get_barrier_semaphore()
        for peer in (X, Y, Z):
            pl.semaphore_signal(barrier_sem, inc=1, device_id=peer,
                                device_id_type=pl.DeviceIdType.MESH)
        pl.semaphore_wait(barrier_sem, 3)

        x_rd = []
        for c in range(NX):
            dst0 = q_me + c * CH if c < NF else q_d + (c - NF) * CH
            r = pltpu.make_async_remote_copy(
                src_ref=sbuf.at[pl.ds(c * CH, CH), :],
                dst_ref=nb_buf.at[pl.ds(dst0, CH), :],
                send_sem=x_s.at[c],
                recv_sem=x_r.at[c],
                device_id=X,
                device_id_type=pl.DeviceIdType.MESH,
            )
            r.start()
            x_rd.append(r)

        def fwd(r0, n, ssem, rsem, dev):
            sl = pl.ds(r0, n)
            r = pltpu.make_async_remote_copy(
                src_ref=nb_buf.at[sl, :],
                dst_ref=nb_buf.at[sl, :],
                send_sem=ssem,
                recv_sem=rsem,
                device_id=dev,
                device_id_type=pl.DeviceIdType.MESH,
            )
            r.start()
            return r

        def compute(r0, n):
            sl = pl.ds(r0, n)
            yv = (partial_ref[0, sl, :] + nb_buf[sl, :].astype(jnp.float32)
                  + resid_ref[sl, :])
            ms = jnp.mean(yv * yv, axis=-1, keepdims=True)
            out_ref[sl, :] = (
                yv * lax.rsqrt(ms + 1e-6) * gamma_ref[...]
            ).astype(jnp.bfloat16)

        yf_rd = []
        zf_rd = []
        for c in range(NF):
            x_rd[c].wait_recv()
            yf_rd.append(fwd(q_me + c * CH, CH, yf_s.at[c], yf_r.at[c], Y))
            zf_rd.append(fwd(q_me + c * CH, CH, zf_s.at[c], zf_r.at[c], Z))

        yr_rd = []
        for i in range(NB):
            zf_rd[NA + i].wait_recv()
            yr_rd.append(fwd(q_z + DA + i * CH, CH, yr_s.at[i], yr_r.at[i], Y))
        zr_rd = []
        for i in range(NC):
            yf_rd[NF - NC + i].wait_recv()
            zr_rd.append(fwd(q_y + DA + DB + i * CH, CH, zr_s.at[i], zr_r.at[i], Z))

        compute(q_me, Q)
        compute(q_z + DA, DB)
        compute(q_y + DA + DB, DC)
        for c in range(NA):
            zf_rd[c].wait_recv()
        compute(q_z, DA)
        for c in range(NA + NB, NF):
            zf_rd[c].wait_recv()
        compute(q_z + DA + DB, DC)
        for c in range(NF - NC):
            yf_rd[c].wait_recv()
        compute(q_y, DA + DB)
        for c in range(NF, NX):
            x_rd[c].wait_recv()
        compute(q_d, DA)
        for r in yr_rd:
            r.wait_recv()
        compute(q_d + DA, DB)
        for r in zr_rd:
            r.wait_recv()
        compute(q_d + DA + DB, DC)

        for r in x_rd + yf_rd + zf_rd + yr_rd + zr_rd:
            r.wait_send()

    return pl.pallas_call(
        body,
        out_shape=jax.ShapeDtypeStruct((m, d), jnp.bfloat16),
        in_specs=[
            pl.BlockSpec(memory_space=pltpu.VMEM),
            pl.BlockSpec(memory_space=pltpu.VMEM),
            pl.BlockSpec(memory_space=pltpu.VMEM),
        ],
        out_specs=pl.BlockSpec(memory_space=pltpu.VMEM),
        scratch_shapes=[
            pltpu.VMEM((SB, d), jnp.bfloat16),
            pltpu.VMEM((m, d), jnp.bfloat16),
            pltpu.SemaphoreType.DMA((NX,)),
            pltpu.SemaphoreType.DMA((NX,)),
            pltpu.SemaphoreType.DMA((NF,)),
            pltpu.SemaphoreType.DMA((NF,)),
            pltpu.SemaphoreType.DMA((NF,)),
            pltpu.SemaphoreType.DMA((NF,)),
            pltpu.SemaphoreType.DMA((NB,)),
            pltpu.SemaphoreType.DMA((NB,)),
            pltpu.SemaphoreType.DMA((NC,)),
            pltpu.SemaphoreType.DMA((NC,)),
        ],
        compiler_params=pltpu.CompilerParams(collective_id=0),
    )(partial, resid, gamma2)
